# baseline (speedup 1.0000x reference)
import jax
import jax.numpy as jnp
from jax.experimental import pallas as pl

_BR = 128
_BM = 512


def _quant_body(w_ref, cen_ref, cent_ref, out_ref):
    w = w_ref[...]
    cen = cen_ref[...]
    cent = cent_ref[...]
    kk = cen.shape[0]
    br = w.shape[0]
    vs = cen.shape[1]
    nb = w.shape[1] // vs
    bv = br * nb
    wt = w.T
    v = wt.reshape(nb, vs, br).transpose(1, 0, 2).reshape(vs, bv)
    n2 = jnp.sum(v * v, axis=0, keepdims=True)
    inv = 1.0 / (jnp.sqrt(n2) + 1e-8)
    nv = v * inv
    c2 = jnp.sum(cen * cen, axis=1, keepdims=True)
    s = jax.lax.dot_general(-2.0 * cen, nv, (((1,), (0,)), ((), ())),
                            preferred_element_type=jnp.float32)
    e = c2 + s
    idx = jnp.argmin(e, axis=0).reshape(1, bv).astype(jnp.int32)
    half = 128
    idm = jnp.where(idx < half, idx, idx - half)
    idxb = jnp.broadcast_to(idm, (vs, bv))
    a_lo = jnp.take_along_axis(cent[:, :half], idxb, axis=1)
    a_hi = jnp.take_along_axis(cent[:, half:], idxb, axis=1)
    assigned = jnp.where(idx < half, a_lo, a_hi)
    num = jnp.sum(v * assigned, axis=0, keepdims=True)
    den = jnp.sum(assigned * assigned, axis=0, keepdims=True) + 1e-8
    q = assigned * (num / den)
    out_ref[...] = q.reshape(vs, nb, br).transpose(1, 0, 2).reshape(w.shape[1], br)


def _mm_body(x_ref, wt_ref, b_ref, o_ref):
    o_ref[...] = jnp.dot(x_ref[...], wt_ref[...],
                         preferred_element_type=jnp.float32) + b_ref[...]


def kernel(x, W, b, centroids):
    dout, din = W.shape
    vs = centroids.shape[1]
    kk = centroids.shape[0]

    wqt = pl.pallas_call(
        _quant_body,
        grid=(dout // _BR,),
        in_specs=[
            pl.BlockSpec((_BR, din), lambda i: (i, 0)),
            pl.BlockSpec((kk, vs), lambda i: (0, 0)),
            pl.BlockSpec((vs, kk), lambda i: (0, 0)),
        ],
        out_specs=pl.BlockSpec((din, _BR), lambda i: (0, i)),
        out_shape=jax.ShapeDtypeStruct((din, dout), jnp.float32),
    )(W, centroids, centroids.T)

    xm = x.reshape(-1, din)
    y = pl.pallas_call(
        _mm_body,
        grid=(xm.shape[0] // _BM,),
        in_specs=[
            pl.BlockSpec((_BM, din), lambda i: (i, 0)),
            pl.BlockSpec((din, dout), lambda i: (0, 0)),
            pl.BlockSpec((1, dout), lambda i: (0, 0)),
        ],
        out_specs=pl.BlockSpec((_BM, dout), lambda i: (i, 0)),
        out_shape=jax.ShapeDtypeStruct((xm.shape[0], dout), jnp.float32),
    )(xm, wqt, b.reshape(1, dout))
    return y.reshape(x.shape)


# speedup vs baseline: 1.0347x; 1.0347x over previous
import jax
import jax.numpy as jnp
from jax.experimental import pallas as pl

_BR = 128
_BM = 1024


def _quant_body(w_ref, cen_ref, cent_ref, out_ref):
    w = w_ref[...]
    cen = cen_ref[...]
    cent = cent_ref[...]
    kk = cen.shape[0]
    br = w.shape[0]
    vs = cen.shape[1]
    nb = w.shape[1] // vs
    bv = br * nb
    wt = w.T
    v = wt.reshape(nb, vs, br).transpose(1, 0, 2).reshape(vs, bv)
    n2 = jnp.sum(v * v, axis=0, keepdims=True)
    inv = 1.0 / (jnp.sqrt(n2) + 1e-8)
    nv = v * inv
    c2 = jnp.sum(cen * cen, axis=1, keepdims=True)
    s = jax.lax.dot_general(-2.0 * cen, nv, (((1,), (0,)), ((), ())),
                            preferred_element_type=jnp.float32)
    e = c2 + s
    idx = jnp.argmin(e, axis=0).reshape(1, bv).astype(jnp.int32)
    half = 128
    idm = jnp.where(idx < half, idx, idx - half)
    idxb = jnp.broadcast_to(idm, (vs, bv))
    a_lo = jnp.take_along_axis(cent[:, :half], idxb, axis=1)
    a_hi = jnp.take_along_axis(cent[:, half:], idxb, axis=1)
    assigned = jnp.where(idx < half, a_lo, a_hi)
    num = jnp.sum(v * assigned, axis=0, keepdims=True)
    den = jnp.sum(assigned * assigned, axis=0, keepdims=True) + 1e-8
    q = assigned * (num / den)
    out_ref[...] = q.reshape(vs, nb, br).transpose(1, 0, 2).reshape(w.shape[1], br)


def _mm_body(x_ref, wt_ref, b_ref, o_ref):
    o_ref[...] = jnp.dot(x_ref[...], wt_ref[...],
                         preferred_element_type=jnp.float32) + b_ref[...]


def kernel(x, W, b, centroids):
    dout, din = W.shape
    vs = centroids.shape[1]
    kk = centroids.shape[0]

    wqt = pl.pallas_call(
        _quant_body,
        grid=(dout // _BR,),
        in_specs=[
            pl.BlockSpec((_BR, din), lambda i: (i, 0)),
            pl.BlockSpec((kk, vs), lambda i: (0, 0)),
            pl.BlockSpec((vs, kk), lambda i: (0, 0)),
        ],
        out_specs=pl.BlockSpec((din, _BR), lambda i: (0, i)),
        out_shape=jax.ShapeDtypeStruct((din, dout), jnp.float32),
    )(W, centroids, centroids.T)

    xm = x.reshape(-1, din)
    y = pl.pallas_call(
        _mm_body,
        grid=(xm.shape[0] // _BM,),
        in_specs=[
            pl.BlockSpec((_BM, din), lambda i: (i, 0)),
            pl.BlockSpec((din, dout), lambda i: (0, 0)),
            pl.BlockSpec((1, dout), lambda i: (0, 0)),
        ],
        out_specs=pl.BlockSpec((_BM, dout), lambda i: (i, 0)),
        out_shape=jax.ShapeDtypeStruct((xm.shape[0], dout), jnp.float32),
    )(xm, wqt, b.reshape(1, dout))
    return y.reshape(x.shape)


# speedup vs baseline: 1.0858x; 1.0494x over previous
import jax
import jax.numpy as jnp
from jax.experimental import pallas as pl
from jax.experimental.pallas import tpu as pltpu

_BR = 128
_BM = 1024
_NQ = 8


def _fused_body(x_ref, w_ref, cen_ref, cent_ref, b_ref, y_ref, wqt_scr):
    i = pl.program_id(0)

    @pl.when(i < _NQ)
    def _quantize():
        w = w_ref[...]
        cen = cen_ref[...]
        cent = cent_ref[...]
        br = w.shape[0]
        vs = cen.shape[1]
        nb = w.shape[1] // vs
        bv = br * nb
        wt = w.T
        v = wt.reshape(nb, vs, br).transpose(1, 0, 2).reshape(vs, bv)
        n2 = jnp.sum(v * v, axis=0, keepdims=True)
        inv = 1.0 / (jnp.sqrt(n2) + 1e-8)
        nv = v * inv
        c2 = jnp.sum(cen * cen, axis=1, keepdims=True)
        s = jax.lax.dot_general(-2.0 * cen, nv, (((1,), (0,)), ((), ())),
                                preferred_element_type=jnp.float32)
        e = c2 + s
        idx = jnp.argmin(e, axis=0).reshape(1, bv).astype(jnp.int32)
        half = 128
        idm = jnp.where(idx < half, idx, idx - half)
        idxb = jnp.broadcast_to(idm, (vs, bv))
        a_lo = jnp.take_along_axis(cent[:, :half], idxb, axis=1)
        a_hi = jnp.take_along_axis(cent[:, half:], idxb, axis=1)
        assigned = jnp.where(idx < half, a_lo, a_hi)
        num = jnp.sum(v * assigned, axis=0, keepdims=True)
        den = jnp.sum(assigned * assigned, axis=0, keepdims=True) + 1e-8
        q = assigned * (num / den)
        wqt_scr[:, pl.ds(i * br, br)] = (
            q.reshape(vs, nb, br).transpose(1, 0, 2).reshape(w.shape[1], br))

    @pl.when(i >= _NQ)
    def _matmul():
        y_ref[...] = jnp.dot(x_ref[...], wqt_scr[...],
                             preferred_element_type=jnp.float32) + b_ref[...]


def kernel(x, W, b, centroids):
    dout, din = W.shape
    vs = centroids.shape[1]
    kk = centroids.shape[0]
    xm = x.reshape(-1, din)
    nm = xm.shape[0] // _BM

    y = pl.pallas_call(
        _fused_body,
        grid=(_NQ + nm,),
        in_specs=[
            pl.BlockSpec((_BM, din), lambda i: (jnp.maximum(i - _NQ, 0), 0)),
            pl.BlockSpec((_BR, din), lambda i: (jnp.minimum(i, _NQ - 1), 0)),
            pl.BlockSpec((kk, vs), lambda i: (0, 0)),
            pl.BlockSpec((vs, kk), lambda i: (0, 0)),
            pl.BlockSpec((1, dout), lambda i: (0, 0)),
        ],
        out_specs=pl.BlockSpec((_BM, dout), lambda i: (jnp.maximum(i - _NQ, 0), 0)),
        out_shape=jax.ShapeDtypeStruct((xm.shape[0], dout), jnp.float32),
        scratch_shapes=[pltpu.VMEM((din, dout), jnp.float32)],
    )(xm, W, centroids, centroids.T, b.reshape(1, dout))
    return y.reshape(x.shape)
